# baseline (speedup 1.0000x reference)
import functools

import jax
import jax.numpy as jnp
from jax import lax
from jax.experimental import pallas as pl
from jax.experimental.pallas import tpu as pltpu
from jax.experimental.pallas import tpu_sc as plsc

N = 10000
G = 200
A = 128
H = 128

NW = 32
NPW = 320
NTAIL = N - (NW - 1) * NPW
SECPT = 384
TRASHB = G * A
ACCSZ = TRASHB + NW * SECPT
ZSTRIPE = TRASHB // 16

_mesh = plsc.VectorSubcoreMesh(core_axis_name="c", subcore_axis_name="s")


@functools.partial(
    pl.kernel,
    mesh=_mesh,
    out_type=jax.ShapeDtypeStruct((2, ACCSZ), jnp.float32),
    scratch_types=[
        pltpu.VMEM((SECPT,), jnp.int32),
        pltpu.VMEM((SECPT,), jnp.int32),
        pltpu.VMEM((3, 128), jnp.int32),
        pltpu.VMEM((3, 128), jnp.float32),
        pltpu.VMEM((ZSTRIPE,), jnp.float32),
        pltpu.VMEM_SHARED((ACCSZ,), jnp.float32),
        pltpu.SemaphoreType.DMA,
        pltpu.SemaphoreType.DMA,
    ],
)
def _reduce_k(batch_hbm, h_hbm, out_hbm,
              bt_v, h_v, idx_st, val_st, z_v, acc_sh, sem_in, sem_sc):
    c = lax.axis_index("c")
    s = lax.axis_index("s")
    wid = s * 2 + c
    base = wid * NPW

    @pl.when(wid != NW - 1)
    def _():
        pltpu.async_copy(batch_hbm.at[pl.ds(base, NPW)],
                         bt_v.at[pl.ds(0, NPW)], sem_in)
        pltpu.async_copy(h_hbm.at[pl.ds(base, NPW)],
                         h_v.at[pl.ds(0, NPW)], sem_in)

    @pl.when(wid == NW - 1)
    def _():
        pltpu.async_copy(batch_hbm.at[pl.ds((NW - 1) * NPW, NTAIL)],
                         bt_v.at[pl.ds(0, NTAIL)], sem_in)
        pltpu.async_copy(h_hbm.at[pl.ds((NW - 1) * NPW, NTAIL)],
                         h_v.at[pl.ds(0, NTAIL)], sem_in)

    zero16 = jnp.zeros((16,), jnp.float32)

    def _zb(k, _):
        z_v[pl.ds(k * 16, 16)] = zero16
        return _
    lax.fori_loop(0, ZSTRIPE // 16, _zb, None)
    pltpu.sync_copy(z_v, acc_sh.at[pl.ds(s * ZSTRIPE, ZSTRIPE)])

    lane = lax.iota(jnp.int32, 16)
    ones = jnp.full((16,), 1.0, jnp.float32)
    trash0 = TRASHB + wid * SECPT

    @pl.when(wid != NW - 1)
    def _():
        pltpu.make_async_copy(batch_hbm.at[pl.ds(base, NPW)],
                              bt_v.at[pl.ds(0, NPW)], sem_in).wait()
        pltpu.make_async_copy(h_hbm.at[pl.ds(base, NPW)],
                              h_v.at[pl.ds(0, NPW)], sem_in).wait()

    @pl.when(wid == NW - 1)
    def _():
        pltpu.make_async_copy(batch_hbm.at[pl.ds((NW - 1) * NPW, NTAIL)],
                              bt_v.at[pl.ds(0, NTAIL)], sem_in).wait()
        pltpu.make_async_copy(h_hbm.at[pl.ds((NW - 1) * NPW, NTAIL)],
                              h_v.at[pl.ds(0, NTAIL)], sem_in).wait()

    for r in range(3):
        def _sb(k, _, r=r):
            ent = r * 128 + k * 16
            sl = pl.ds(ent, 16)
            nid = base + ent + lane
            ok = jnp.logical_and(nid < N, ent + lane < NPW)
            idx = jnp.where(ok, bt_v[sl] * A + h_v[sl], trash0 + ent + lane)
            col = pl.ds(k * 16, 16)
            idx_st[r, col] = idx
            val_st[r, col] = ones
            return _
        lax.fori_loop(0, 8, _sb, None)

    plsc.subcore_barrier()

    sc_cp = [pltpu.async_copy(val_st.at[rr], acc_sh.at[idx_st.at[rr]],
                              sem_sc, add=True) for rr in range(3)]
    for cp in sc_cp:
        cp.wait()

    plsc.subcore_barrier()

    @pl.when(s == 0)
    def _():
        pltpu.sync_copy(acc_sh, out_hbm.at[c])


def _possum_body(bat_ref, pos_ref, poss_ref):
    f32 = jnp.float32
    bf = jnp.bfloat16
    one_b = jnp.bfloat16(1.0)
    zero_b = jnp.bfloat16(0.0)
    bat_b = bat_ref[...].astype(bf)
    gcol = lax.broadcasted_iota(jnp.int32, (G, 1), 0).astype(bf)
    oh = jnp.where(bat_b[None, :] == gcol, one_b, zero_b)
    pch = pos_ref[...]
    phi = pch.astype(bf)
    plo = (pch - phi.astype(f32)).astype(bf)
    phl = jnp.concatenate([phi, plo, jnp.zeros((N, 2), bf)], axis=1)
    poss_ref[...] = jnp.dot(oh, phl, preferred_element_type=f32)


_possum = pl.pallas_call(
    _possum_body,
    out_shape=jax.ShapeDtypeStruct((G, 8), jnp.float32),
)


def _chain_body(stats_ref, poss_ref, bat_ref, emb_ref, win_ref, bin_ref,
                wc1_ref, bc1_ref, wc2_ref, bc2_ref, wc3_ref, bc3_ref,
                wo1_ref, bo1_ref, wo2_ref, bo2_ref, out_ref):
    f32 = jnp.float32
    bf = jnp.bfloat16
    st = stats_ref[0] + stats_ref[1]
    hist = st[:G, :]
    cnt = jnp.sum(hist, axis=1, keepdims=True)
    inv = jnp.where(cnt > 0, 1.0 / cnt, 0.0)

    one_b = jnp.bfloat16(1.0)
    zero_b = jnp.bfloat16(0.0)
    bat_b = bat_ref[...].astype(bf)
    gcol = lax.broadcasted_iota(jnp.int32, (G, 1), 0).astype(bf)
    oh = jnp.where(bat_b[None, :] == gcol, one_b, zero_b)

    poss6 = poss_ref[...]
    poss = poss6[:, :3] + poss6[:, 3:6]

    wemb = jnp.dot(emb_ref[...], win_ref[3:, :], preferred_element_type=f32)
    msum = (jnp.dot(hist, wemb, preferred_element_type=f32)
            + jnp.dot(poss, win_ref[:3, :], preferred_element_type=f32))
    x = msum * inv + bin_ref[...]
    for w_ref, b_ref in ((wc1_ref, bc1_ref), (wc2_ref, bc2_ref),
                         (wc3_ref, bc3_ref), (wo1_ref, bo1_ref)):
        x = jax.nn.silu(jnp.dot(x, w_ref[...], preferred_element_type=f32)
                        + b_ref[...])
    r = jnp.dot(x, wo2_ref[...], preferred_element_type=f32) + bo2_ref[...]

    rhi = r.astype(bf)
    rlo = (r - rhi.astype(f32)).astype(bf)
    rhl = jnp.concatenate([rhi, rlo, jnp.zeros((G, 2), bf)], axis=1)
    och = lax.dot_general(rhl, oh, (((0,), (0,)), ((), ())),
                          preferred_element_type=f32)
    out_ref[...] = och[:3, :] + och[3:6, :]


_chain = pl.pallas_call(
    _chain_body,
    out_shape=jax.ShapeDtypeStruct((3, N), jnp.float32),
)


def kernel(pos, h, batch, atom_embed, W_in, b_in, Wc1, bc1, Wc2, bc2,
           Wc3, bc3, Wo1, bo1, Wo2, bo2):
    bat32 = batch.astype(jnp.int32)
    stats = _reduce_k(bat32, h.astype(jnp.int32))
    poss = _possum(bat32, pos)
    stats = stats.reshape(2, ACCSZ // 128, 128)
    outt = _chain(stats, poss, bat32, atom_embed, W_in,
                  b_in, Wc1, bc1, Wc2, bc2, Wc3, bc3, Wo1, bo1, Wo2, bo2)
    return outt.T


# speedup vs baseline: 1.2389x; 1.2389x over previous
import functools

import jax
import jax.numpy as jnp
from jax import lax
from jax.experimental import pallas as pl
from jax.experimental.pallas import tpu as pltpu
from jax.experimental.pallas import tpu_sc as plsc

N = 10000
G = 200
A = 128
H = 128

NW = 32
NPW = 320
NTAIL = N - (NW - 1) * NPW
SECPT = 384
TRASHB = G * A
ACCSZ = TRASHB + NW * SECPT
ZSTRIPE = TRASHB // 16

_mesh = plsc.VectorSubcoreMesh(core_axis_name="c", subcore_axis_name="s")


@functools.partial(
    pl.kernel,
    mesh=_mesh,
    out_type=[jax.ShapeDtypeStruct((ACCSZ,), jnp.float32),
              jax.ShapeDtypeStruct((ACCSZ,), jnp.float32)],
    scratch_types=[
        pltpu.VMEM((SECPT,), jnp.int32),
        pltpu.VMEM((SECPT,), jnp.int32),
        pltpu.VMEM((3, 128), jnp.int32),
        pltpu.VMEM((3, 128), jnp.float32),
        pltpu.VMEM((ZSTRIPE,), jnp.float32),
        pltpu.VMEM_SHARED((ACCSZ,), jnp.float32),
        pltpu.SemaphoreType.DMA,
        pltpu.SemaphoreType.DMA,
    ],
)
def _reduce_k(batch_hbm, h_hbm, out0_hbm, out1_hbm,
              bt_v, h_v, idx_st, val_st, z_v, acc_sh, sem_in, sem_sc):
    c = lax.axis_index("c")
    s = lax.axis_index("s")
    wid = s * 2 + c
    base = wid * NPW

    @pl.when(wid != NW - 1)
    def _():
        pltpu.async_copy(batch_hbm.at[pl.ds(base, NPW)],
                         bt_v.at[pl.ds(0, NPW)], sem_in)
        pltpu.async_copy(h_hbm.at[pl.ds(base, NPW)],
                         h_v.at[pl.ds(0, NPW)], sem_in)

    @pl.when(wid == NW - 1)
    def _():
        pltpu.async_copy(batch_hbm.at[pl.ds((NW - 1) * NPW, NTAIL)],
                         bt_v.at[pl.ds(0, NTAIL)], sem_in)
        pltpu.async_copy(h_hbm.at[pl.ds((NW - 1) * NPW, NTAIL)],
                         h_v.at[pl.ds(0, NTAIL)], sem_in)

    zero16 = jnp.zeros((16,), jnp.float32)

    def _zb(k, _):
        z_v[pl.ds(k * 16, 16)] = zero16
        return _
    lax.fori_loop(0, ZSTRIPE // 16, _zb, None)
    pltpu.sync_copy(z_v, acc_sh.at[pl.ds(s * ZSTRIPE, ZSTRIPE)])

    lane = lax.iota(jnp.int32, 16)
    ones = jnp.full((16,), 1.0, jnp.float32)
    trash0 = TRASHB + wid * SECPT

    @pl.when(wid != NW - 1)
    def _():
        pltpu.make_async_copy(batch_hbm.at[pl.ds(base, NPW)],
                              bt_v.at[pl.ds(0, NPW)], sem_in).wait()
        pltpu.make_async_copy(h_hbm.at[pl.ds(base, NPW)],
                              h_v.at[pl.ds(0, NPW)], sem_in).wait()

    @pl.when(wid == NW - 1)
    def _():
        pltpu.make_async_copy(batch_hbm.at[pl.ds((NW - 1) * NPW, NTAIL)],
                              bt_v.at[pl.ds(0, NTAIL)], sem_in).wait()
        pltpu.make_async_copy(h_hbm.at[pl.ds((NW - 1) * NPW, NTAIL)],
                              h_v.at[pl.ds(0, NTAIL)], sem_in).wait()

    for r in range(3):
        def _sb(k, _, r=r):
            ent = r * 128 + k * 16
            sl = pl.ds(ent, 16)
            nid = base + ent + lane
            ok = jnp.logical_and(nid < N, ent + lane < NPW)
            idx = jnp.where(ok, bt_v[sl] * A + h_v[sl], trash0 + ent + lane)
            col = pl.ds(k * 16, 16)
            idx_st[r, col] = idx
            val_st[r, col] = ones
            return _
        lax.fori_loop(0, 8, _sb, None)

    plsc.subcore_barrier()

    sc_cp = [pltpu.async_copy(val_st.at[rr], acc_sh.at[idx_st.at[rr]],
                              sem_sc, add=True) for rr in range(3)]
    for cp in sc_cp:
        cp.wait()

    plsc.subcore_barrier()

    @pl.when(jnp.logical_and(s == 0, c == 0))
    def _():
        pltpu.sync_copy(acc_sh, out0_hbm)

    @pl.when(jnp.logical_and(s == 0, c == 1))
    def _():
        pltpu.sync_copy(acc_sh, out1_hbm)


def _possum_body(bat_ref, post_ref, poss_ref):
    f32 = jnp.float32
    bf = jnp.bfloat16
    one_b = jnp.bfloat16(1.0)
    zero_b = jnp.bfloat16(0.0)
    bat_b = bat_ref[...].astype(bf)
    gcol = lax.broadcasted_iota(jnp.int32, (G, 1), 0).astype(bf)
    oh = jnp.where(bat_b[None, :] == gcol, one_b, zero_b)
    pch = post_ref[...]
    phi = pch.astype(bf)
    plo = (pch - phi.astype(f32)).astype(bf)
    phl = jnp.concatenate([phi, plo, jnp.zeros((2, N), bf)], axis=0)
    poss_ref[...] = lax.dot_general(phl, oh, (((1,), (1,)), ((), ())),
                                    preferred_element_type=f32)


_possum = pl.pallas_call(
    _possum_body,
    out_shape=jax.ShapeDtypeStruct((8, G), jnp.float32),
)


def _chain_body(st0_ref, st1_ref, poss_ref, bat_ref, emb_ref, win_ref,
                bin_ref, wc1_ref, bc1_ref, wc2_ref, bc2_ref, wc3_ref,
                bc3_ref, wo1_ref, bo1_ref, wo2_ref, bo2_ref, out_ref):
    f32 = jnp.float32
    bf = jnp.bfloat16
    st = st0_ref[...] + st1_ref[...]
    hist = st[:G, :]
    cnt = jnp.sum(hist, axis=1, keepdims=True)
    inv = jnp.where(cnt > 0, 1.0 / cnt, 0.0)

    one_b = jnp.bfloat16(1.0)
    zero_b = jnp.bfloat16(0.0)
    bat_b = bat_ref[...].astype(bf)
    gcol = lax.broadcasted_iota(jnp.int32, (G, 1), 0).astype(bf)
    oh = jnp.where(bat_b[None, :] == gcol, one_b, zero_b)

    poss6 = poss_ref[...]
    pp = poss6[:3, :] + poss6[3:6, :]

    wemb = jnp.dot(emb_ref[...], win_ref[3:, :], preferred_element_type=f32)
    msum = (jnp.dot(hist, wemb, preferred_element_type=f32)
            + lax.dot_general(pp, win_ref[:3, :], (((0,), (0,)), ((), ())),
                              preferred_element_type=f32))
    x = msum * inv + bin_ref[...]
    for w_ref, b_ref in ((wc1_ref, bc1_ref), (wc2_ref, bc2_ref),
                         (wc3_ref, bc3_ref), (wo1_ref, bo1_ref)):
        x = jax.nn.silu(jnp.dot(x, w_ref[...], preferred_element_type=f32)
                        + b_ref[...])
    r = jnp.dot(x, wo2_ref[...], preferred_element_type=f32) + bo2_ref[...]

    rhi = r.astype(bf)
    rlo = (r - rhi.astype(f32)).astype(bf)
    rhl = jnp.concatenate([rhi, rlo, jnp.zeros((G, 2), bf)], axis=1)
    och = lax.dot_general(rhl, oh, (((0,), (0,)), ((), ())),
                          preferred_element_type=f32)
    out_ref[...] = och[:3, :] + och[3:6, :]


_chain = pl.pallas_call(
    _chain_body,
    out_shape=jax.ShapeDtypeStruct((3, N), jnp.float32),
)


def kernel(pos, h, batch, atom_embed, W_in, b_in, Wc1, bc1, Wc2, bc2,
           Wc3, bc3, Wo1, bo1, Wo2, bo2):
    bat32 = batch.astype(jnp.int32)
    st0, st1 = _reduce_k(bat32, h.astype(jnp.int32))
    poss = _possum(bat32, pos.T)
    outt = _chain(st0.reshape(ACCSZ // 128, 128),
                  st1.reshape(ACCSZ // 128, 128), poss, bat32, atom_embed,
                  W_in, b_in, Wc1, bc1, Wc2, bc2, Wc3, bc3, Wo1, bo1,
                  Wo2, bo2)
    return outt.T


# speedup vs baseline: 1.3049x; 1.0533x over previous
import functools

import jax
import jax.numpy as jnp
from jax import lax
from jax.experimental import pallas as pl
from jax.experimental.pallas import tpu as pltpu
from jax.experimental.pallas import tpu_sc as plsc

N = 10000
G = 200
A = 128
H = 128

NW = 16
NPW = 640
NTAIL = N - (NW - 1) * NPW
SECPT = 768
SROWS = SECPT // 128
TRASHB = G * A
ACCSZ = TRASHB + NW * SECPT
ZSTRIPE = TRASHB // 16

_mesh = plsc.VectorSubcoreMesh(core_axis_name="c", subcore_axis_name="s",
                               num_cores=1)


@functools.partial(
    pl.kernel,
    mesh=_mesh,
    out_type=jax.ShapeDtypeStruct((ACCSZ,), jnp.float32),
    scratch_types=[
        pltpu.VMEM((SECPT,), jnp.int32),
        pltpu.VMEM((SECPT,), jnp.int32),
        pltpu.VMEM((SROWS, 128), jnp.int32),
        pltpu.VMEM((SROWS, 128), jnp.float32),
        pltpu.VMEM((ZSTRIPE,), jnp.float32),
        pltpu.VMEM_SHARED((ACCSZ,), jnp.float32),
        pltpu.SemaphoreType.DMA,
        pltpu.SemaphoreType.DMA,
    ],
)
def _reduce_k(batch_hbm, h_hbm, out_hbm,
              bt_v, h_v, idx_st, val_st, z_v, acc_sh, sem_in, sem_sc):
    s = lax.axis_index("s")
    wid = s
    base = wid * NPW

    @pl.when(wid != NW - 1)
    def _():
        pltpu.async_copy(batch_hbm.at[pl.ds(base, NPW)],
                         bt_v.at[pl.ds(0, NPW)], sem_in)
        pltpu.async_copy(h_hbm.at[pl.ds(base, NPW)],
                         h_v.at[pl.ds(0, NPW)], sem_in)

    @pl.when(wid == NW - 1)
    def _():
        pltpu.async_copy(batch_hbm.at[pl.ds((NW - 1) * NPW, NTAIL)],
                         bt_v.at[pl.ds(0, NTAIL)], sem_in)
        pltpu.async_copy(h_hbm.at[pl.ds((NW - 1) * NPW, NTAIL)],
                         h_v.at[pl.ds(0, NTAIL)], sem_in)

    zero16 = jnp.zeros((16,), jnp.float32)

    def _zb(k, _):
        z_v[pl.ds(k * 16, 16)] = zero16
        return _
    lax.fori_loop(0, ZSTRIPE // 16, _zb, None)
    pltpu.sync_copy(z_v, acc_sh.at[pl.ds(s * ZSTRIPE, ZSTRIPE)])

    lane = lax.iota(jnp.int32, 16)
    ones = jnp.full((16,), 1.0, jnp.float32)
    trash0 = TRASHB + wid * SECPT

    @pl.when(wid != NW - 1)
    def _():
        pltpu.make_async_copy(batch_hbm.at[pl.ds(base, NPW)],
                              bt_v.at[pl.ds(0, NPW)], sem_in).wait()
        pltpu.make_async_copy(h_hbm.at[pl.ds(base, NPW)],
                              h_v.at[pl.ds(0, NPW)], sem_in).wait()

    @pl.when(wid == NW - 1)
    def _():
        pltpu.make_async_copy(batch_hbm.at[pl.ds((NW - 1) * NPW, NTAIL)],
                              bt_v.at[pl.ds(0, NTAIL)], sem_in).wait()
        pltpu.make_async_copy(h_hbm.at[pl.ds((NW - 1) * NPW, NTAIL)],
                              h_v.at[pl.ds(0, NTAIL)], sem_in).wait()

    for r in range(SROWS):
        def _sb(k, _, r=r):
            ent = r * 128 + k * 16
            sl = pl.ds(ent, 16)
            nid = base + ent + lane
            ok = jnp.logical_and(nid < N, ent + lane < NPW)
            idx = jnp.where(ok, bt_v[sl] * A + h_v[sl], trash0 + ent + lane)
            col = pl.ds(k * 16, 16)
            idx_st[r, col] = idx
            val_st[r, col] = ones
            return _
        lax.fori_loop(0, 8, _sb, None)

    plsc.subcore_barrier()

    sc_cp = [pltpu.async_copy(val_st.at[rr], acc_sh.at[idx_st.at[rr]],
                              sem_sc, add=True) for rr in range(SROWS)]
    for cp in sc_cp:
        cp.wait()

    plsc.subcore_barrier()

    @pl.when(s == 0)
    def _():
        pltpu.sync_copy(acc_sh, out_hbm)


def _possum_body(bat_ref, post_ref, poss_ref):
    f32 = jnp.float32
    bf = jnp.bfloat16
    one_b = jnp.bfloat16(1.0)
    zero_b = jnp.bfloat16(0.0)
    bat_b = bat_ref[...].astype(bf)
    gcol = lax.broadcasted_iota(jnp.int32, (G, 1), 0).astype(bf)
    oh = jnp.where(bat_b[None, :] == gcol, one_b, zero_b)
    pch = post_ref[...]
    phi = pch.astype(bf)
    plo = (pch - phi.astype(f32)).astype(bf)
    phl = jnp.concatenate([phi, plo, jnp.zeros((2, N), bf)], axis=0)
    poss_ref[...] = lax.dot_general(phl, oh, (((1,), (1,)), ((), ())),
                                    preferred_element_type=f32)


_possum = pl.pallas_call(
    _possum_body,
    out_shape=jax.ShapeDtypeStruct((8, G), jnp.float32),
)


def _chain_body(st0_ref, poss_ref, bat_ref, emb_ref, win_ref,
                bin_ref, wc1_ref, bc1_ref, wc2_ref, bc2_ref, wc3_ref,
                bc3_ref, wo1_ref, bo1_ref, wo2_ref, bo2_ref, out_ref):
    f32 = jnp.float32
    bf = jnp.bfloat16
    st = st0_ref[...]
    hist = st[:G, :]
    cnt = jnp.sum(hist, axis=1, keepdims=True)
    inv = jnp.where(cnt > 0, 1.0 / cnt, 0.0)

    one_b = jnp.bfloat16(1.0)
    zero_b = jnp.bfloat16(0.0)
    bat_b = bat_ref[...].astype(bf)
    gcol = lax.broadcasted_iota(jnp.int32, (G, 1), 0).astype(bf)
    oh = jnp.where(bat_b[None, :] == gcol, one_b, zero_b)

    poss6 = poss_ref[...]
    pp = poss6[:3, :] + poss6[3:6, :]

    wemb = jnp.dot(emb_ref[...], win_ref[3:, :], preferred_element_type=f32)
    msum = (jnp.dot(hist, wemb, preferred_element_type=f32)
            + lax.dot_general(pp, win_ref[:3, :], (((0,), (0,)), ((), ())),
                              preferred_element_type=f32))
    x = msum * inv + bin_ref[...]
    for w_ref, b_ref in ((wc1_ref, bc1_ref), (wc2_ref, bc2_ref),
                         (wc3_ref, bc3_ref), (wo1_ref, bo1_ref)):
        x = jax.nn.silu(jnp.dot(x, w_ref[...], preferred_element_type=f32)
                        + b_ref[...])
    r = jnp.dot(x, wo2_ref[...], preferred_element_type=f32) + bo2_ref[...]

    rhi = r.astype(bf)
    rlo = (r - rhi.astype(f32)).astype(bf)
    rhl = jnp.concatenate([rhi, rlo, jnp.zeros((G, 2), bf)], axis=1)
    och = lax.dot_general(rhl, oh, (((0,), (0,)), ((), ())),
                          preferred_element_type=f32)
    out_ref[...] = och[:3, :] + och[3:6, :]


_chain = pl.pallas_call(
    _chain_body,
    out_shape=jax.ShapeDtypeStruct((3, N), jnp.float32),
)


def kernel(pos, h, batch, atom_embed, W_in, b_in, Wc1, bc1, Wc2, bc2,
           Wc3, bc3, Wo1, bo1, Wo2, bo2):
    bat32 = batch.astype(jnp.int32)
    st0 = _reduce_k(bat32, h.astype(jnp.int32))
    poss = _possum(bat32, pos.T)
    outt = _chain(st0.reshape(ACCSZ // 128, 128), poss, bat32, atom_embed,
                  W_in, b_in, Wc1, bc1, Wc2, bc2, Wc3, bc3, Wo1, bo1,
                  Wo2, bo2)
    return outt.T
